# Initial kernel scaffold; baseline (speedup 1.0000x reference)
import functools

import jax
import jax.numpy as jnp
from jax import lax
from jax.experimental import pallas as pl
from jax.experimental.pallas import tpu as pltpu
from jax.experimental.pallas import tpu_sc as plsc

_EMBED_DIM = 32


def _make_gather(B, D, num_workers, chunk):
    assert B % (num_workers * chunk) == 0
    b_per_w = B // num_workers
    n_chunks = b_per_w // chunk
    mesh = plsc.VectorSubcoreMesh(core_axis_name="c", subcore_axis_name="s")

    @functools.partial(
        pl.kernel,
        mesh=mesh,
        out_type=jax.ShapeDtypeStruct((B, D), jnp.float32),
        scratch_types=[
            pltpu.VMEM((chunk,), jnp.int32),
            pltpu.VMEM((chunk, D), jnp.float32),
            pltpu.SemaphoreType.DMA,
        ],
    )
    def gather_kernel(idx_hbm, table_hbm, out_hbm, idx_v, rows_v, sem):
        num_cores = lax.axis_size("c")
        wid = lax.axis_index("s") * num_cores + lax.axis_index("c")
        base = wid * b_per_w

        def body(i, carry):
            off = base + i * chunk
            pltpu.sync_copy(idx_hbm.at[pl.ds(off, chunk)], idx_v)
            pltpu.async_copy(table_hbm.at[idx_v], rows_v, sem).wait()
            pltpu.sync_copy(rows_v, out_hbm.at[pl.ds(off, chunk)])
            return carry

        lax.fori_loop(0, n_chunks, body, 0)

    return gather_kernel


@jax.jit
def kernel(x, W):
    B = x.shape[0] * x.shape[1]
    xf = x.reshape(B).astype(jnp.int32)
    out = _make_gather(B, _EMBED_DIM, 32, 1024)(xf, W)
    return out.reshape(x.shape[0], x.shape[1], _EMBED_DIM)


# speedup vs baseline: 1.1012x; 1.1012x over previous
import functools

import jax
import jax.numpy as jnp
from jax import lax
from jax.experimental import pallas as pl
from jax.experimental.pallas import tpu as pltpu
from jax.experimental.pallas import tpu_sc as plsc

_EMBED_DIM = 32


def _make_gather(B, D, num_workers, chunk):
    assert B % (num_workers * chunk) == 0
    b_per_w = B // num_workers
    n_chunks = b_per_w // chunk
    mesh = plsc.VectorSubcoreMesh(core_axis_name="c", subcore_axis_name="s")

    @functools.partial(
        pl.kernel,
        mesh=mesh,
        out_type=jax.ShapeDtypeStruct((B, D), jnp.float32),
        compiler_params=pltpu.CompilerParams(use_tc_tiling_on_sc=False),
        scratch_types=[
            pltpu.VMEM((chunk,), jnp.int32),
            pltpu.VMEM((chunk, D), jnp.float32),
            pltpu.SemaphoreType.DMA,
        ],
    )
    def gather_kernel(idx_hbm, table_hbm, out_hbm, idx_v, rows_v, sem):
        num_cores = lax.axis_size("c")
        wid = lax.axis_index("s") * num_cores + lax.axis_index("c")
        base = wid * b_per_w

        def body(i, carry):
            off = base + i * chunk
            pltpu.sync_copy(idx_hbm.at[pl.ds(off, chunk)], idx_v)
            pltpu.async_copy(table_hbm.at[idx_v], rows_v, sem).wait()
            pltpu.sync_copy(rows_v, out_hbm.at[pl.ds(off, chunk)])
            return carry

        lax.fori_loop(0, n_chunks, body, 0)

    return gather_kernel


@jax.jit
def kernel(x, W):
    B = x.shape[0] * x.shape[1]
    xf = x.reshape(B).astype(jnp.int32)
    out = _make_gather(B, _EMBED_DIM, 32, 1024)(xf, W)
    return out.reshape(x.shape[0], x.shape[1], _EMBED_DIM)


# speedup vs baseline: 1.1118x; 1.0097x over previous
import functools

import jax
import jax.numpy as jnp
from jax import lax
from jax.experimental import pallas as pl
from jax.experimental.pallas import tpu as pltpu
from jax.experimental.pallas import tpu_sc as plsc

_EMBED_DIM = 32
_NBUF = 4
_LOOKAHEAD = 2


def _make_gather(B, D, num_workers, chunk):
    assert B % (num_workers * chunk) == 0
    b_per_w = B // num_workers
    n = b_per_w // chunk
    assert n % _NBUF == 0 and n >= 3 * _NBUF
    mesh = plsc.VectorSubcoreMesh(core_axis_name="c", subcore_axis_name="s")

    @functools.partial(
        pl.kernel,
        mesh=mesh,
        out_type=jax.ShapeDtypeStruct((B, D), jnp.float32),
        compiler_params=pltpu.CompilerParams(use_tc_tiling_on_sc=False),
        scratch_types=(
            [pltpu.VMEM((b_per_w,), jnp.int32)]
            + [pltpu.VMEM((chunk, D), jnp.float32) for _ in range(_NBUF)]
            + [pltpu.SemaphoreType.DMA for _ in range(2 * _NBUF)]
        ),
    )
    def gather_kernel(idx_hbm, table_hbm, out_hbm, idx_all, *bufs_and_sems):
        rows = bufs_and_sems[:_NBUF]
        gsem = bufs_and_sems[_NBUF : 2 * _NBUF]
        ssem = bufs_and_sems[2 * _NBUF :]

        num_cores = lax.axis_size("c")
        wid = lax.axis_index("s") * num_cores + lax.axis_index("c")
        base = wid * b_per_w

        pltpu.sync_copy(idx_hbm.at[pl.ds(base, b_per_w)], idx_all)

        def gather_desc(i, b):
            return pltpu.make_async_copy(
                table_hbm.at[idx_all.at[pl.ds(i * chunk, chunk)]], rows[b], gsem[b]
            )

        def store_desc(i, b):
            return pltpu.make_async_copy(
                rows[b], out_hbm.at[pl.ds(base + i * chunk, chunk)], ssem[b]
            )

        def step(i, b):
            j = i + _LOOKAHEAD
            if isinstance(j, int) and j >= n:
                pass
            else:
                bj = (b + _LOOKAHEAD) % _NBUF
                if not (isinstance(j, int) and j < _NBUF):
                    store_desc(j - _NBUF, bj).wait()
                gather_desc(j, bj).start()
            gather_desc(i, b).wait()
            store_desc(i, b).start()

        for j in range(_LOOKAHEAD):
            gather_desc(j, j % _NBUF).start()
        for i in range(_NBUF):
            step(i, i % _NBUF)

        def group(g, carry):
            for b in range(_NBUF):
                step(g * _NBUF + b, b)
            return carry

        lax.fori_loop(1, n // _NBUF - 1, group, 0)

        for i in range(n - _NBUF, n):
            step(i, i % _NBUF)
        for b in range(_NBUF):
            i = n - _NBUF + b
            store_desc(i, b).wait()

    return gather_kernel


@jax.jit
def kernel(x, W):
    B = x.shape[0] * x.shape[1]
    xf = x.reshape(B).astype(jnp.int32)
    out = _make_gather(B, _EMBED_DIM, 32, 512)(xf, W)
    return out.reshape(x.shape[0], x.shape[1], _EMBED_DIM)


# speedup vs baseline: 1.1124x; 1.0005x over previous
import functools

import jax
import jax.numpy as jnp
from jax import lax
from jax.experimental import pallas as pl
from jax.experimental.pallas import tpu as pltpu
from jax.experimental.pallas import tpu_sc as plsc

_EMBED_DIM = 32
_NBUF = 8
_LOOKAHEAD = 6


def _make_gather(B, D, num_workers, chunk):
    assert B % (num_workers * chunk) == 0
    b_per_w = B // num_workers
    n = b_per_w // chunk
    assert n % _NBUF == 0 and n >= 3 * _NBUF
    mesh = plsc.VectorSubcoreMesh(core_axis_name="c", subcore_axis_name="s")

    @functools.partial(
        pl.kernel,
        mesh=mesh,
        out_type=jax.ShapeDtypeStruct((B, D), jnp.float32),
        compiler_params=pltpu.CompilerParams(use_tc_tiling_on_sc=False),
        scratch_types=(
            [pltpu.VMEM((b_per_w,), jnp.int32)]
            + [pltpu.VMEM((chunk, D), jnp.float32) for _ in range(_NBUF)]
            + [pltpu.SemaphoreType.DMA for _ in range(2 * _NBUF)]
        ),
    )
    def gather_kernel(idx_hbm, table_hbm, out_hbm, idx_all, *bufs_and_sems):
        rows = bufs_and_sems[:_NBUF]
        gsem = bufs_and_sems[_NBUF : 2 * _NBUF]
        ssem = bufs_and_sems[2 * _NBUF :]

        num_cores = lax.axis_size("c")
        wid = lax.axis_index("s") * num_cores + lax.axis_index("c")
        base = wid * b_per_w

        pltpu.sync_copy(idx_hbm.at[pl.ds(base, b_per_w)], idx_all)

        def gather_desc(i, b):
            return pltpu.make_async_copy(
                table_hbm.at[idx_all.at[pl.ds(i * chunk, chunk)]], rows[b], gsem[b]
            )

        def store_desc(i, b):
            return pltpu.make_async_copy(
                rows[b], out_hbm.at[pl.ds(base + i * chunk, chunk)], ssem[b]
            )

        def step(i, b):
            j = i + _LOOKAHEAD
            if isinstance(j, int) and j >= n:
                pass
            else:
                bj = (b + _LOOKAHEAD) % _NBUF
                if not (isinstance(j, int) and j < _NBUF):
                    store_desc(j - _NBUF, bj).wait()
                gather_desc(j, bj).start()
            gather_desc(i, b).wait()
            store_desc(i, b).start()

        for j in range(_LOOKAHEAD):
            gather_desc(j, j % _NBUF).start()
        for i in range(_NBUF):
            step(i, i % _NBUF)

        def group(g, carry):
            for b in range(_NBUF):
                step(g * _NBUF + b, b)
            return carry

        lax.fori_loop(1, n // _NBUF - 1, group, 0)

        for i in range(n - _NBUF, n):
            step(i, i % _NBUF)
        for b in range(_NBUF):
            i = n - _NBUF + b
            store_desc(i, b).wait()

    return gather_kernel


@jax.jit
def kernel(x, W):
    B = x.shape[0] * x.shape[1]
    xf = x.reshape(B).astype(jnp.int32)
    out = _make_gather(B, _EMBED_DIM, 32, 256)(xf, W)
    return out.reshape(x.shape[0], x.shape[1], _EMBED_DIM)


# speedup vs baseline: 3.6035x; 3.2395x over previous
import functools

import jax
import jax.numpy as jnp
from jax import lax
from jax.experimental import pallas as pl
from jax.experimental.pallas import tpu as pltpu
from jax.experimental.pallas import tpu_sc as plsc

_EMBED_DIM = 32
_NBUF = 8
_LOOKAHEAD = 6


def _make_gather(B, D, num_workers, chunk):
    assert B % (num_workers * chunk) == 0
    b_per_w = B // num_workers
    n = b_per_w // chunk
    assert n % _NBUF == 0 and n >= 3 * _NBUF
    mesh = plsc.VectorSubcoreMesh(core_axis_name="c", subcore_axis_name="s")

    @functools.partial(
        pl.kernel,
        mesh=mesh,
        out_type=jax.ShapeDtypeStruct((B, D), jnp.float32),
        compiler_params=pltpu.CompilerParams(use_tc_tiling_on_sc=False),
        scratch_types=(
            [pltpu.VMEM((b_per_w,), jnp.int32)]
            + [pltpu.VMEM((chunk, D), jnp.float32) for _ in range(_NBUF)]
            + [pltpu.SemaphoreType.DMA for _ in range(2 * _NBUF)]
        ),
    )
    def gather_kernel(idx_hbm, table_hbm, out_hbm, idx_all, *bufs_and_sems):
        rows = bufs_and_sems[:_NBUF]
        gsem = bufs_and_sems[_NBUF : 2 * _NBUF]
        ssem = bufs_and_sems[2 * _NBUF :]

        num_cores = lax.axis_size("c")
        wid = lax.axis_index("s") * num_cores + lax.axis_index("c")
        base = wid * b_per_w

        pltpu.sync_copy(idx_hbm.at[pl.ds(base, b_per_w)], idx_all)

        def gather_desc(i, b):
            return pltpu.make_async_copy(
                table_hbm.at[idx_all.at[pl.ds(i * chunk, chunk)]], rows[b], gsem[b]
            )

        def store_desc(i, b):
            return pltpu.make_async_copy(
                rows[b], out_hbm.at[pl.ds(base + i * chunk, chunk)], ssem[b]
            )

        def step(i, b):
            j = i + _LOOKAHEAD
            if isinstance(j, int) and j >= n:
                pass
            else:
                bj = (b + _LOOKAHEAD) % _NBUF
                if not (isinstance(j, int) and j < _NBUF):
                    store_desc(j - _NBUF, bj).wait()
                gather_desc(j, bj).start()
            gather_desc(i, b).wait()
            store_desc(i, b).start()

        for j in range(_LOOKAHEAD):
            gather_desc(j, j % _NBUF).start()
        for i in range(_NBUF):
            step(i, i % _NBUF)

        def group(g, carry):
            for b in range(_NBUF):
                step(g * _NBUF + b, b)
            return carry

        lax.fori_loop(1, n // _NBUF - 1, group, 0)

        for i in range(n - _NBUF, n):
            step(i, i % _NBUF)
        for b in range(_NBUF):
            i = n - _NBUF + b
            store_desc(i, b).wait()

    return gather_kernel


def _transpose_rows(G, F, Bdim, D, bk):
    nb = Bdim // bk

    def body(g_ref, o_ref):
        o_ref[...] = jnp.swapaxes(g_ref[...], 0, 1)

    return pl.pallas_call(
        body,
        grid=(F, nb),
        in_specs=[pl.BlockSpec((bk, D), lambda f, b: (f * nb + b, 0))],
        out_specs=pl.BlockSpec((D, bk), lambda f, b: (f, b)),
        out_shape=jax.ShapeDtypeStruct((F * D, Bdim), jnp.float32),
    )(G)


@jax.jit
def kernel(x, W):
    Bdim, F = x.shape
    D = _EMBED_DIM
    xt = jnp.swapaxes(x, 0, 1).reshape(Bdim * F).astype(jnp.int32)
    G = _make_gather(Bdim * F, D, 32, 256)(xt, W)
    out_t = _transpose_rows(G, F, Bdim, D, 2048)
    return out_t.reshape(F, D, Bdim).transpose(2, 0, 1)


# speedup vs baseline: 4.1807x; 1.1602x over previous
import functools

import jax
import jax.numpy as jnp
from jax import lax
from jax.experimental import pallas as pl
from jax.experimental.pallas import tpu as pltpu
from jax.experimental.pallas import tpu_sc as plsc

_EMBED_DIM = 32
_NBUF = 8
_LOOKAHEAD = 6


def _make_gather(B, D, num_workers, chunk):
    assert B % (num_workers * chunk) == 0
    b_per_w = B // num_workers
    n = b_per_w // chunk
    assert n % _NBUF == 0 and n >= 3 * _NBUF
    mesh = plsc.VectorSubcoreMesh(core_axis_name="c", subcore_axis_name="s")

    @functools.partial(
        pl.kernel,
        mesh=mesh,
        out_type=jax.ShapeDtypeStruct((B, D), jnp.float32),
        compiler_params=pltpu.CompilerParams(use_tc_tiling_on_sc=False),
        scratch_types=(
            [pltpu.VMEM((b_per_w,), jnp.int32)]
            + [pltpu.VMEM((chunk, D), jnp.float32) for _ in range(_NBUF)]
            + [pltpu.SemaphoreType.DMA for _ in range(2 * _NBUF)]
        ),
    )
    def gather_kernel(idx_hbm, table_hbm, out_hbm, idx_all, *bufs_and_sems):
        rows = bufs_and_sems[:_NBUF]
        gsem = bufs_and_sems[_NBUF : 2 * _NBUF]
        ssem = bufs_and_sems[2 * _NBUF :]

        num_cores = lax.axis_size("c")
        wid = lax.axis_index("s") * num_cores + lax.axis_index("c")
        base = wid * b_per_w

        pltpu.sync_copy(idx_hbm.at[pl.ds(base, b_per_w)], idx_all)

        def gather_desc(i, b):
            return pltpu.make_async_copy(
                table_hbm.at[idx_all.at[pl.ds(i * chunk, chunk)]], rows[b], gsem[b]
            )

        def store_desc(i, b):
            return pltpu.make_async_copy(
                rows[b], out_hbm.at[pl.ds(base + i * chunk, chunk)], ssem[b]
            )

        def step(i, b):
            j = i + _LOOKAHEAD
            if isinstance(j, int) and j >= n:
                pass
            else:
                bj = (b + _LOOKAHEAD) % _NBUF
                if not (isinstance(j, int) and j < _NBUF):
                    store_desc(j - _NBUF, bj).wait()
                gather_desc(j, bj).start()
            gather_desc(i, b).wait()
            store_desc(i, b).start()

        for j in range(_LOOKAHEAD):
            gather_desc(j, j % _NBUF).start()
        for i in range(_NBUF):
            step(i, i % _NBUF)

        def group(g, carry):
            for b in range(_NBUF):
                step(g * _NBUF + b, b)
            return carry

        lax.fori_loop(1, n // _NBUF - 1, group, 0)

        for i in range(n - _NBUF, n):
            step(i, i % _NBUF)
        for b in range(_NBUF):
            i = n - _NBUF + b
            store_desc(i, b).wait()

    return gather_kernel


def _transpose_g128(G128, FG, Bdim, bk):
    nb = Bdim // bk

    def body(g_ref, o_ref):
        o_ref[...] = jnp.swapaxes(g_ref[...], 0, 1)

    return pl.pallas_call(
        body,
        grid=(FG, nb),
        in_specs=[pl.BlockSpec((bk, 128), lambda fg, c: (fg * nb + c, 0))],
        out_specs=pl.BlockSpec((128, bk), lambda fg, c: (fg, c)),
        out_shape=jax.ShapeDtypeStruct((FG * 128, Bdim), jnp.float32),
    )(G128)


@jax.jit
def kernel(x, W):
    Bdim, F = x.shape
    D = _EMBED_DIM
    FG = F // 4
    idx = x.reshape(Bdim, FG, 4).transpose(1, 0, 2).reshape(-1).astype(jnp.int32)
    G = _make_gather(Bdim * F, D, 32, 256)(idx, W)
    G128 = G.reshape(FG * Bdim, 4 * D)
    out_t = _transpose_g128(G128, FG, Bdim, 512)
    return out_t.reshape(F, D, Bdim).transpose(2, 0, 1)
